# baseline (speedup 1.0000x reference)
import functools

import jax
import jax.numpy as jnp
from jax import lax
from jax.experimental import pallas as pl
from jax.experimental.pallas import tpu as pltpu
from jax.experimental.pallas import tpu_sc as plsc

BATCH = 16384
FEAT = 8
NC = 2
NS = 16
NW = NC * NS
LANES = 16
TILE = 128


def _sc_gather_body(b_per_w, users_hbm, tracks_hbm, utbl_hbm, ttbl_hbm,
                    u_out_hbm, t_out_hbm, uidx_v, tidx_v,
                    utiles_v, ttiles_v, urows_v, trows_v, sem_idx, sem_data):
    wid = lax.axis_index("s") * NC + lax.axis_index("c")
    base = wid * b_per_w
    cp_u = pltpu.make_async_copy(users_hbm.at[pl.ds(base, b_per_w)], uidx_v,
                                 sem_idx)
    cp_t = pltpu.make_async_copy(tracks_hbm.at[pl.ds(base, b_per_w)], tidx_v,
                                 sem_idx)
    cp_u.start()
    cp_t.start()
    cp_u.wait()
    cp_t.wait()

    lanes = lax.iota(jnp.int32, LANES)

    def starts_of(vec):
        return vec & ~jnp.int32(TILE - 1)

    def loop(g, carry):
        uvec = uidx_v[pl.ds(g * LANES, LANES)]
        tvec = tidx_v[pl.ds(g * LANES, LANES)]
        us = starts_of(uvec)
        ts = starts_of(tvec)
        for j in range(LANES):
            su = pl.multiple_of(us[j], TILE)
            st = pl.multiple_of(ts[j], TILE)
            pltpu.make_async_copy(utbl_hbm.at[:, pl.ds(su, TILE)],
                                  utiles_v.at[j], sem_data).start()
            pltpu.make_async_copy(ttbl_hbm.at[:, pl.ds(st, TILE)],
                                  ttiles_v.at[j], sem_data).start()
        pltpu.make_async_copy(utbl_hbm.at[:, pl.ds(0, TILE * LANES)],
                              utiles_v, sem_data).wait()
        pltpu.make_async_copy(ttbl_hbm.at[:, pl.ds(0, TILE * LANES)],
                              ttiles_v, sem_data).wait()
        ulane = uvec - us
        tlane = tvec - ts
        for f in range(FEAT):
            fvec = jnp.full((LANES,), f, jnp.int32)
            uvals = plsc.load_gather(utiles_v, [lanes, fvec, ulane])
            tvals = plsc.load_gather(ttiles_v, [lanes, fvec, tlane])
            urows_v[pl.ds(f * b_per_w + g * LANES, LANES)] = uvals
            trows_v[pl.ds(f * b_per_w + g * LANES, LANES)] = tvals
        return carry

    lax.fori_loop(0, b_per_w // LANES, loop, 0)

    for f in range(FEAT):
        pltpu.sync_copy(
            urows_v.at[pl.ds(f * b_per_w, b_per_w)],
            u_out_hbm.at[f, pl.ds(base, b_per_w)])
        pltpu.sync_copy(
            trows_v.at[pl.ds(f * b_per_w, b_per_w)],
            t_out_hbm.at[f, pl.ds(base, b_per_w)])


def _sc_gather(batch, users, tracks, utbl_t, ttbl_t):
    b_per_w = batch // NW
    mesh = plsc.VectorSubcoreMesh(core_axis_name="c", subcore_axis_name="s",
                                  num_cores=NC, num_subcores=NS)
    return pl.kernel(
        functools.partial(_sc_gather_body, b_per_w),
        out_type=[
            jax.ShapeDtypeStruct((FEAT, batch), jnp.float32),
            jax.ShapeDtypeStruct((FEAT, batch), jnp.float32),
        ],
        mesh=mesh,
        compiler_params=pltpu.CompilerParams(needs_layout_passes=False),
        scratch_types=[
            pltpu.VMEM((b_per_w,), jnp.int32),
            pltpu.VMEM((b_per_w,), jnp.int32),
            pltpu.VMEM((LANES, FEAT, TILE), jnp.float32),
            pltpu.VMEM((LANES, FEAT, TILE), jnp.float32),
            pltpu.VMEM((FEAT * b_per_w,), jnp.float32),
            pltpu.VMEM((FEAT * b_per_w,), jnp.float32),
            pltpu.SemaphoreType.DMA,
            pltpu.SemaphoreType.DMA,
        ],
    )(users, tracks, utbl_t, ttbl_t)


BLK = 2048


def _mlp_body(u_ref, t_ref, w1a_ref, w1b_ref, b1_ref, w2_ref, b2_ref,
              w3_ref, b3_ref, out_ref):
    cdims = (((0,), (0,)), ((), ()))
    h = lax.dot_general(u_ref[...], w1a_ref[...], cdims,
                        preferred_element_type=jnp.float32)
    h += lax.dot_general(t_ref[...], w1b_ref[...], cdims,
                         preferred_element_type=jnp.float32)
    h = jax.nn.relu(h + b1_ref[...])
    h = jnp.dot(h, w2_ref[...], preferred_element_type=jnp.float32)
    h = jax.nn.relu(h + b2_ref[...])
    o = jnp.dot(h, w3_ref[...], preferred_element_type=jnp.float32)
    out_ref[...] = jax.nn.sigmoid(o + b3_ref[...])


def _mlp(batch, u_et, t_et, W1, b1, W2, b2, W3, b3):
    w1a, w1b = W1[:FEAT], W1[FEAT:]
    grid = batch // BLK
    return pl.pallas_call(
        _mlp_body,
        grid=(grid,),
        in_specs=[
            pl.BlockSpec((FEAT, BLK), lambda i: (0, i)),
            pl.BlockSpec((FEAT, BLK), lambda i: (0, i)),
            pl.BlockSpec((FEAT, 64), lambda i: (0, 0)),
            pl.BlockSpec((FEAT, 64), lambda i: (0, 0)),
            pl.BlockSpec((1, 64), lambda i: (0, 0)),
            pl.BlockSpec((64, 32), lambda i: (0, 0)),
            pl.BlockSpec((1, 32), lambda i: (0, 0)),
            pl.BlockSpec((32, 1), lambda i: (0, 0)),
            pl.BlockSpec((1, 1), lambda i: (0, 0)),
        ],
        out_specs=pl.BlockSpec((BLK, 1), lambda i: (i, 0)),
        out_shape=jax.ShapeDtypeStruct((batch, 1), jnp.float32),
    )(u_et, t_et, w1a, w1b, b1.reshape(1, 64), W2, b2.reshape(1, 32),
      W3, b3.reshape(1, 1))


def kernel(users, tracks, user_table, track_table, W1, b1, W2, b2, W3, b3):
    utbl_t = user_table.T
    ttbl_t = track_table.T
    half = BATCH // 2
    outs = []
    for h in range(2):
        sl = slice(h * half, (h + 1) * half)
        u_et, t_et = _sc_gather(half, users[sl], tracks[sl], utbl_t, ttbl_t)
        outs.append(_mlp(half, u_et, t_et, W1, b1, W2, b2, W3, b3))
    return jnp.concatenate(outs, axis=0)


# speedup vs baseline: 1.1112x; 1.1112x over previous
import jax
import jax.numpy as jnp
from jax import lax
from jax.experimental import pallas as pl
from jax.experimental.pallas import tpu as pltpu
from jax.experimental.pallas import tpu_sc as plsc

BATCH = 16384
FEAT = 8
NC = 2
NS = 16
NW = NC * NS
B_PER_W = BATCH // NW
LANES = 16
N_GROUPS = B_PER_W // LANES
TILE = 128


def _sc_gather_body(users_hbm, tracks_hbm, utbl_hbm, ttbl_hbm,
                    u_out_hbm, t_out_hbm, uidx_v, tidx_v,
                    utiles_v, ttiles_v, urows_v, trows_v, sem_idx, sem_data):
    wid = lax.axis_index("s") * NC + lax.axis_index("c")
    base = wid * B_PER_W
    cp_u = pltpu.make_async_copy(users_hbm.at[pl.ds(base, B_PER_W)], uidx_v,
                                 sem_idx)
    cp_t = pltpu.make_async_copy(tracks_hbm.at[pl.ds(base, B_PER_W)], tidx_v,
                                 sem_idx)
    cp_u.start()
    cp_t.start()
    cp_u.wait()
    cp_t.wait()

    lanes = lax.iota(jnp.int32, LANES)

    def starts_of(vec):
        return vec & ~jnp.int32(TILE - 1)

    def loop(g, carry):
        uvec = uidx_v[pl.ds(g * LANES, LANES)]
        tvec = tidx_v[pl.ds(g * LANES, LANES)]
        us = starts_of(uvec)
        ts = starts_of(tvec)
        for j in range(LANES):
            su = pl.multiple_of(us[j], TILE)
            st = pl.multiple_of(ts[j], TILE)
            pltpu.make_async_copy(utbl_hbm.at[:, pl.ds(su, TILE)],
                                  utiles_v.at[j], sem_data).start()
            pltpu.make_async_copy(ttbl_hbm.at[:, pl.ds(st, TILE)],
                                  ttiles_v.at[j], sem_data).start()
        pltpu.make_async_copy(utbl_hbm.at[:, pl.ds(0, TILE * LANES)],
                              utiles_v, sem_data).wait()
        pltpu.make_async_copy(ttbl_hbm.at[:, pl.ds(0, TILE * LANES)],
                              ttiles_v, sem_data).wait()
        ulane = uvec - us
        tlane = tvec - ts
        for f in range(FEAT):
            fvec = jnp.full((LANES,), f, jnp.int32)
            uvals = plsc.load_gather(utiles_v, [lanes, fvec, ulane])
            tvals = plsc.load_gather(ttiles_v, [lanes, fvec, tlane])
            urows_v[pl.ds(f * B_PER_W + g * LANES, LANES)] = uvals
            trows_v[pl.ds(f * B_PER_W + g * LANES, LANES)] = tvals
        return carry

    lax.fori_loop(0, N_GROUPS, loop, 0)

    for f in range(FEAT):
        pltpu.sync_copy(
            urows_v.at[pl.ds(f * B_PER_W, B_PER_W)],
            u_out_hbm.at[f, pl.ds(base, B_PER_W)])
        pltpu.sync_copy(
            trows_v.at[pl.ds(f * B_PER_W, B_PER_W)],
            t_out_hbm.at[f, pl.ds(base, B_PER_W)])


def _sc_gather(users, tracks, utbl_t, ttbl_t):
    mesh = plsc.VectorSubcoreMesh(core_axis_name="c", subcore_axis_name="s",
                                  num_cores=NC, num_subcores=NS)
    return pl.kernel(
        _sc_gather_body,
        out_type=[
            jax.ShapeDtypeStruct((FEAT, BATCH), jnp.float32),
            jax.ShapeDtypeStruct((FEAT, BATCH), jnp.float32),
        ],
        mesh=mesh,
        compiler_params=pltpu.CompilerParams(needs_layout_passes=False),
        scratch_types=[
            pltpu.VMEM((B_PER_W,), jnp.int32),
            pltpu.VMEM((B_PER_W,), jnp.int32),
            pltpu.VMEM((LANES, FEAT, TILE), jnp.float32),
            pltpu.VMEM((LANES, FEAT, TILE), jnp.float32),
            pltpu.VMEM((FEAT * B_PER_W,), jnp.float32),
            pltpu.VMEM((FEAT * B_PER_W,), jnp.float32),
            pltpu.SemaphoreType.DMA,
            pltpu.SemaphoreType.DMA,
        ],
    )(users, tracks, utbl_t, ttbl_t)


def _mlp_body(u_ref, t_ref, w1a_ref, w1b_ref, b1_ref, w2_ref, b2_ref,
              w3_ref, b3_ref, out_ref):
    cdims = (((0,), (0,)), ((), ()))
    h = lax.dot_general(u_ref[...], w1a_ref[...], cdims,
                        preferred_element_type=jnp.float32)
    h += lax.dot_general(t_ref[...], w1b_ref[...], cdims,
                         preferred_element_type=jnp.float32)
    h = jax.nn.relu(h + b1_ref[...])
    h = jnp.dot(h, w2_ref[...], preferred_element_type=jnp.float32)
    h = jax.nn.relu(h + b2_ref[...])
    o = lax.dot_general(w3_ref[...], h, (((0,), (1,)), ((), ())),
                        preferred_element_type=jnp.float32)
    out_ref[...] = jax.nn.sigmoid(o + b3_ref[...])


def _mlp(u_et, t_et, W1, b1, W2, b2, W3, b3):
    w1a, w1b = W1[:FEAT], W1[FEAT:]
    return pl.pallas_call(
        _mlp_body,
        out_shape=jax.ShapeDtypeStruct((1, BATCH), jnp.float32),
    )(u_et, t_et, w1a, w1b, b1.reshape(1, 64), W2, b2.reshape(1, 32),
      W3.reshape(32, 1), b3.reshape(1, 1))


def kernel(users, tracks, user_table, track_table, W1, b1, W2, b2, W3, b3):
    u_et, t_et = _sc_gather(users, tracks, user_table.T, track_table.T)
    out_t = _mlp(u_et, t_et, W1, b1, W2, b2, W3, b3)
    return out_t.reshape(BATCH, 1)


# speedup vs baseline: 1.2354x; 1.1118x over previous
import jax
import jax.numpy as jnp
from jax import lax
from jax.experimental import pallas as pl
from jax.experimental.pallas import tpu as pltpu
from jax.experimental.pallas import tpu_sc as plsc

BATCH = 16384
FEAT = 8
NC = 2
NS = 16
NW = NC * NS
B_PER_W = BATCH // NW
LANES = 16
N_GROUPS = B_PER_W // LANES
TILE = 128


def _sc_gather_body(users_hbm, tracks_hbm, utbl_hbm, ttbl_hbm,
                    u_out_hbm, t_out_hbm, uidx_v, tidx_v,
                    utiles_v, ttiles_v, urows_v, trows_v, sem_idx, sem_data):
    wid = lax.axis_index("s") * NC + lax.axis_index("c")
    base = wid * B_PER_W
    cp_u = pltpu.make_async_copy(users_hbm.at[pl.ds(base, B_PER_W)], uidx_v,
                                 sem_idx)
    cp_t = pltpu.make_async_copy(tracks_hbm.at[pl.ds(base, B_PER_W)], tidx_v,
                                 sem_idx)
    cp_u.start()
    cp_t.start()
    cp_u.wait()
    cp_t.wait()

    lanes = lax.iota(jnp.int32, LANES)

    def starts_of(vec):
        return vec & ~jnp.int32(TILE - 1)

    def loop(g, carry):
        vecs = []
        for h in range(2):
            uvec = uidx_v[pl.ds((2 * g + h) * LANES, LANES)]
            tvec = tidx_v[pl.ds((2 * g + h) * LANES, LANES)]
            us = starts_of(uvec)
            ts = starts_of(tvec)
            vecs.append((uvec, tvec, us, ts))
            for j in range(LANES):
                su = pl.multiple_of(us[j], TILE)
                st = pl.multiple_of(ts[j], TILE)
                pltpu.make_async_copy(utbl_hbm.at[:, pl.ds(su, TILE)],
                                      utiles_v.at[h * LANES + j],
                                      sem_data).start()
                pltpu.make_async_copy(ttbl_hbm.at[:, pl.ds(st, TILE)],
                                      ttiles_v.at[h * LANES + j],
                                      sem_data).start()
        pltpu.make_async_copy(utbl_hbm.at[:, pl.ds(0, TILE * 2 * LANES)],
                              utiles_v, sem_data).wait()
        pltpu.make_async_copy(ttbl_hbm.at[:, pl.ds(0, TILE * 2 * LANES)],
                              ttiles_v, sem_data).wait()
        for h in range(2):
            uvec, tvec, us, ts = vecs[h]
            ulane = uvec - us
            tlane = tvec - ts
            slot = lanes + h * LANES
            for f in range(FEAT):
                fvec = jnp.full((LANES,), f, jnp.int32)
                uvals = plsc.load_gather(utiles_v, [slot, fvec, ulane])
                tvals = plsc.load_gather(ttiles_v, [slot, fvec, tlane])
                b0 = f * B_PER_W + (2 * g + h) * LANES
                urows_v[pl.ds(b0, LANES)] = uvals
                trows_v[pl.ds(b0, LANES)] = tvals
        return carry

    lax.fori_loop(0, N_GROUPS // 2, loop, 0)

    for f in range(FEAT):
        pltpu.sync_copy(
            urows_v.at[pl.ds(f * B_PER_W, B_PER_W)],
            u_out_hbm.at[f, pl.ds(base, B_PER_W)])
        pltpu.sync_copy(
            trows_v.at[pl.ds(f * B_PER_W, B_PER_W)],
            t_out_hbm.at[f, pl.ds(base, B_PER_W)])


def _sc_gather(users, tracks, utbl_t, ttbl_t):
    mesh = plsc.VectorSubcoreMesh(core_axis_name="c", subcore_axis_name="s",
                                  num_cores=NC, num_subcores=NS)
    return pl.kernel(
        _sc_gather_body,
        out_type=[
            jax.ShapeDtypeStruct((FEAT, BATCH), jnp.float32),
            jax.ShapeDtypeStruct((FEAT, BATCH), jnp.float32),
        ],
        mesh=mesh,
        compiler_params=pltpu.CompilerParams(needs_layout_passes=False),
        scratch_types=[
            pltpu.VMEM((B_PER_W,), jnp.int32),
            pltpu.VMEM((B_PER_W,), jnp.int32),
            pltpu.VMEM((2 * LANES, FEAT, TILE), jnp.float32),
            pltpu.VMEM((2 * LANES, FEAT, TILE), jnp.float32),
            pltpu.VMEM((FEAT * B_PER_W,), jnp.float32),
            pltpu.VMEM((FEAT * B_PER_W,), jnp.float32),
            pltpu.SemaphoreType.DMA,
            pltpu.SemaphoreType.DMA,
        ],
    )(users, tracks, utbl_t, ttbl_t)


def _mlp_body(u_ref, t_ref, w1a_ref, w1b_ref, b1_ref, w2_ref, b2_ref,
              w3_ref, b3_ref, out_ref):
    cdims = (((0,), (0,)), ((), ()))
    h = lax.dot_general(u_ref[...], w1a_ref[...], cdims,
                        preferred_element_type=jnp.float32)
    h += lax.dot_general(t_ref[...], w1b_ref[...], cdims,
                         preferred_element_type=jnp.float32)
    h = jax.nn.relu(h + b1_ref[...])
    h = jnp.dot(h, w2_ref[...], preferred_element_type=jnp.float32)
    h = jax.nn.relu(h + b2_ref[...])
    o = lax.dot_general(w3_ref[...], h, (((0,), (1,)), ((), ())),
                        preferred_element_type=jnp.float32)
    out_ref[...] = jax.nn.sigmoid(o + b3_ref[...])


def _mlp(u_et, t_et, W1, b1, W2, b2, W3, b3):
    w1a, w1b = W1[:FEAT], W1[FEAT:]
    return pl.pallas_call(
        _mlp_body,
        out_shape=jax.ShapeDtypeStruct((1, BATCH), jnp.float32),
    )(u_et, t_et, w1a, w1b, b1.reshape(1, 64), W2, b2.reshape(1, 32),
      W3.reshape(32, 1), b3.reshape(1, 1))


def kernel(users, tracks, user_table, track_table, W1, b1, W2, b2, W3, b3):
    u_et, t_et = _sc_gather(users, tracks, user_table.T, track_table.T)
    out_t = _mlp(u_et, t_et, W1, b1, W2, b2, W3, b3)
    return out_t.reshape(BATCH, 1)
